# baseline (speedup 1.0000x reference)
import functools
import math

import jax
import jax.numpy as jnp
from jax import lax
from jax.experimental import pallas as pl
from jax.experimental.pallas import tpu as pltpu
from jax.experimental.pallas import tpu_sc as plsc

N = 10000
E = 320000
ND = 128
ED = 16
H = 8
D = 16
SCALE = 1.0 / math.sqrt(D)

NB = 1000
EB = 2560
G1 = 80
NG1 = 125
G2 = 40
NG2 = 250
NW = 32
EPW = E // NW

_SC_PARAMS = pltpu.CompilerParams(
    needs_layout_passes=False, use_tc_tiling_on_sc=False)


def _tables_body(node_ref, wnq_ref, wnk_ref, wnv_ref, bkd_ref, bqd_ref,
                 bq_ref, bk_ref, bv_ref, a_ref, b_ref, v_ref):
    x = node_ref[...]
    dn = (((1,), (1,)), ((), ()))
    at = lax.dot_general(x, wnq_ref[...], dn,
                         preferred_element_type=jnp.float32) + bq_ref[...]
    bt = lax.dot_general(x, wnk_ref[...], dn,
                         preferred_element_type=jnp.float32) + bk_ref[...]
    vt = lax.dot_general(x, wnv_ref[...], dn,
                         preferred_element_type=jnp.float32) + bv_ref[...]
    ap = jnp.dot(at, bkd_ref[...], preferred_element_type=jnp.float32)
    bp = jnp.dot(bt, bqd_ref[...], preferred_element_type=jnp.float32)
    a_ref[:, :ND] = at
    a_ref[:, ND:] = ap
    b_ref[:, :ND] = bt
    b_ref[:, ND:] = bp
    v_ref[...] = vt


def _make_tables(node, wnq, wnk, wnv, bkd, bqd, bq, bk, bv):
    full = lambda shape: pl.BlockSpec(shape, lambda i: (0,) * len(shape))
    return pl.pallas_call(
        _tables_body,
        grid=(N // NB,),
        in_specs=[
            pl.BlockSpec((NB, ND), lambda i: (i, 0)),
            full((ND, ND)), full((ND, ND)), full((ND, ND)),
            full((ND, ND)), full((ND, ND)),
            full((1, ND)), full((1, ND)), full((1, ND)),
        ],
        out_specs=[
            pl.BlockSpec((NB, 2 * ND), lambda i: (i, 0)),
            pl.BlockSpec((NB, 2 * ND), lambda i: (i, 0)),
            pl.BlockSpec((NB, ND), lambda i: (i, 0)),
        ],
        out_shape=[
            jax.ShapeDtypeStruct((N, 2 * ND), jnp.float32),
            jax.ShapeDtypeStruct((N, 2 * ND), jnp.float32),
            jax.ShapeDtypeStruct((N, ND), jnp.float32),
        ],
    )(node, wnq, wnk, wnv, bkd, bqd, bq, bk, bv)


def _t1_body(edge_ref, weq_ref, wek_ref, wev_ref, ones_ref, eye_ref,
             t1t_ref, et_ref, ev0_ref):
    x = edge_ref[...]
    dn = (((1,), (1,)), ((), ()))
    eq = lax.dot_general(x, weq_ref[...], dn,
                         preferred_element_type=jnp.float32)
    ek = lax.dot_general(x, wek_ref[...], dn,
                         preferred_element_type=jnp.float32)
    t1t_ref[...] = lax.dot_general(ones_ref[...], eq * ek,
                                   (((0,), (1,)), ((), ())),
                                   preferred_element_type=jnp.float32)
    et_ref[...] = lax.dot_general(eye_ref[...], x,
                                  (((1,), (1,)), ((), ())),
                                  preferred_element_type=jnp.float32)
    ev0_ref[...] = lax.dot_general(x, wev_ref[...], dn,
                                   preferred_element_type=jnp.float32)


def _make_t1_ev0(edge, weq, wek, wev, ones8, eye16):
    return pl.pallas_call(
        _t1_body,
        grid=(E // EB,),
        in_specs=[
            pl.BlockSpec((EB, ED), lambda i: (i, 0)),
            pl.BlockSpec((ND, ED), lambda i: (0, 0)),
            pl.BlockSpec((ND, ED), lambda i: (0, 0)),
            pl.BlockSpec((ND, ED), lambda i: (0, 0)),
            pl.BlockSpec((ND, H), lambda i: (0, 0)),
            pl.BlockSpec((ED, ED), lambda i: (0, 0)),
        ],
        out_specs=[
            pl.BlockSpec((H, EB), lambda i: (0, i)),
            pl.BlockSpec((ED, EB), lambda i: (0, i)),
            pl.BlockSpec((EB, ND), lambda i: (i, 0)),
        ],
        out_shape=[
            jax.ShapeDtypeStruct((H, E), jnp.float32),
            jax.ShapeDtypeStruct((ED, E), jnp.float32),
            jax.ShapeDtypeStruct((E, ND), jnp.float32),
        ],
    )(edge, weq, wek, wev, ones8, eye16)


def _sc1_body(a_hbm, b_hbm, et_hbm, t1t_hbm, ei_hbm,
              ex_hbm,
              src_all, dst_all,
              a_rows0, a_rows1, b_rows0, b_rows1, edge_v0, edge_v1,
              t1_v0, t1_v1, ex_v0, ex_v1,
              sem_i0, sem_i1, sem_o0, sem_o1):
    c = lax.axis_index("c")
    s = lax.axis_index("s")
    base = (c * 16 + s) * EPW
    iot = jnp.arange(16, dtype=jnp.int32)
    bufs = ((a_rows0, b_rows0, edge_v0, t1_v0, ex_v0, sem_i0, sem_o0),
            (a_rows1, b_rows1, edge_v1, t1_v1, ex_v1, sem_i1, sem_o1))

    pltpu.sync_copy(ei_hbm.at[0, pl.ds(base, EPW)], src_all)
    pltpu.sync_copy(ei_hbm.at[1, pl.ds(base, EPW)], dst_all)

    def in_copies(g, b):
        ar, br, ev, tv, _, si, _ = bufs[b]
        off = base + g * G1
        return (
            pltpu.make_async_copy(et_hbm.at[:, pl.ds(off, G1)], ev, si),
            pltpu.make_async_copy(t1t_hbm.at[:, pl.ds(off, G1)], tv, si),
            pltpu.make_async_copy(a_hbm.at[src_all.at[pl.ds(g * G1, G1)]],
                                  ar, si),
            pltpu.make_async_copy(b_hbm.at[dst_all.at[pl.ds(g * G1, G1)]],
                                  br, si),
        )

    def issue_in(g, b):
        for cp in in_copies(g, b):
            cp.start()

    def wait_in(g, b):
        for cp in in_copies(g, b):
            cp.wait()

    def out_copy(g, b):
        xv, so = bufs[b][4], bufs[b][6]
        off = base + g * G1
        return pltpu.make_async_copy(xv, ex_hbm.at[:, pl.ds(off, G1)], so)

    def compute(b):
        ar, br, ev, tv, xv = bufs[b][:5]

        def one_edge(e):
            esp = jnp.full((16,), e, jnp.int32)
            edg = plsc.load_gather(ev, [iot, esp])
            sums = []
            for h in range(H):
                qh = ar[e, pl.ds(h * 16, 16)]
                kh = br[e, pl.ds(h * 16, 16)]
                ph = (ar[e, pl.ds(ND + h * 16, 16)]
                      + br[e, pl.ds(ND + h * 16, 16)])
                sums.append(jnp.sum(qh * kh + ph * edg))
            acc = jnp.zeros((16,), jnp.float32)
            for h in range(H):
                acc = acc + jnp.where(iot == h, sums[h], jnp.float32(0.0))
            t1g = plsc.load_gather(tv, [iot & 7, esp])
            plsc.store_scatter(xv, [iot, esp],
                               jnp.exp((acc + t1g) * SCALE))

        def ebody(p, _):
            one_edge(p * 2)
            one_edge(p * 2 + 1)
            return 0
        lax.fori_loop(0, G1 // 2, ebody, 0)

    issue_in(0, 0)
    issue_in(1, 1)

    def pair(g2, _):
        g = g2 * 2
        for b in (0, 1):
            gb = g + b
            wait_in(gb, b)
            @pl.when(gb >= 2)
            def _():
                out_copy(gb - 2, b).wait()
            compute(b)
            out_copy(gb, b).start()
            @pl.when(gb < NG1 - 2)
            def _():
                issue_in(gb + 2, b)
        return 0
    lax.fori_loop(0, (NG1 - 1) // 2, pair, 0)

    gb = NG1 - 1
    wait_in(gb, 0)
    out_copy(gb - 2, 0).wait()
    compute(0)
    out_copy(gb, 0).start()
    out_copy(gb - 1, 1).wait()
    out_copy(gb, 0).wait()


def _run_sc1(a, b, et, t1t, ei):
    mesh = plsc.VectorSubcoreMesh(core_axis_name="c", subcore_axis_name="s")
    kern = functools.partial(
        pl.kernel,
        mesh=mesh,
        out_type=jax.ShapeDtypeStruct((16, E), jnp.float32),
        scratch_types=[
            pltpu.VMEM((EPW,), jnp.int32),
            pltpu.VMEM((EPW,), jnp.int32),
            pltpu.VMEM((G1, 2 * ND), jnp.float32),
            pltpu.VMEM((G1, 2 * ND), jnp.float32),
            pltpu.VMEM((G1, 2 * ND), jnp.float32),
            pltpu.VMEM((G1, 2 * ND), jnp.float32),
            pltpu.VMEM((ED, G1), jnp.float32),
            pltpu.VMEM((ED, G1), jnp.float32),
            pltpu.VMEM((H, G1), jnp.float32),
            pltpu.VMEM((H, G1), jnp.float32),
            pltpu.VMEM((16, G1), jnp.float32),
            pltpu.VMEM((16, G1), jnp.float32),
            pltpu.SemaphoreType.DMA,
            pltpu.SemaphoreType.DMA,
            pltpu.SemaphoreType.DMA,
            pltpu.SemaphoreType.DMA,
        ],
        compiler_params=_SC_PARAMS,
    )(_sc1_body)
    return kern(a, b, et, t1t, ei)


def _sc2_body(vt_hbm, ev0_hbm, ex_hbm, ei_hbm,
              pssum_hbm, pagg_hbm,
              dst_all,
              src_v0, src_v1, ex_v0, ex_v1, ev0_v0, ev0_v1,
              vt_rows0, vt_rows1, ss_v0, ss_v1, zb128, zb16,
              agg_sh, ssum_sh,
              sem_i0, sem_i1, sem_s0, sem_s1):
    c = lax.axis_index("c")
    s = lax.axis_index("s")
    base = (c * 16 + s) * EPW
    iot = jnp.arange(16, dtype=jnp.int32)
    bufs = ((src_v0, ex_v0, ev0_v0, vt_rows0, ss_v0, sem_i0, sem_s0),
            (src_v1, ex_v1, ev0_v1, vt_rows1, ss_v1, sem_i1, sem_s1))

    pltpu.sync_copy(ei_hbm.at[1, pl.ds(base, EPW)], dst_all)

    zero16 = jnp.zeros((16,), jnp.float32)
    def zr(i, _):
        for j in range(8):
            zb128[i, pl.ds(j * 16, 16)] = zero16
        zb16[i, :] = zero16
        return 0
    lax.fori_loop(0, 25, zr, 0)
    def zr2(i, _):
        zb16[i, :] = zero16
        return 0
    lax.fori_loop(25, 200, zr2, 0)
    @pl.when(s < 10)
    def _():
        for k in range(40):
            pltpu.sync_copy(zb128,
                            agg_sh.at[pl.ds(s * 1000 + k * 25, 25), :])
        for k in range(5):
            pltpu.sync_copy(zb16,
                            ssum_sh.at[pl.ds(s * 1000 + k * 200, 200), :])
    plsc.subcore_barrier()

    def in_copies(g, b):
        sv, xv, ov, vr, _, si, _ = bufs[b]
        off = base + g * G2
        return (
            pltpu.make_async_copy(ei_hbm.at[0, pl.ds(off, G2)], sv, si),
            pltpu.make_async_copy(ex_hbm.at[:, pl.ds(off, G2)], xv, si),
            pltpu.make_async_copy(ev0_hbm.at[pl.ds(off, G2), :], ov, si),
            pltpu.make_async_copy(vt_hbm.at[dst_all.at[pl.ds(g * G2, G2)]],
                                  vr, si),
        )

    def issue_in(g, b):
        for cp in in_copies(g, b):
            cp.start()

    def wait_in(g, b):
        for cp in in_copies(g, b):
            cp.wait()

    def scatter_copies(b):
        sv, _, ov, _, sr, _, ss = bufs[b]
        return (
            pltpu.make_async_copy(ov, agg_sh.at[sv], ss),
            pltpu.make_async_copy(sr, ssum_sh.at[sv], ss),
        )

    def compute(b):
        _, xv, ov, vr, sr = bufs[b][:5]

        def ebody(e, _):
            exv = plsc.load_gather(xv, [iot, jnp.full((16,), e, jnp.int32)])
            sr[e, :] = exv
            for h in range(H):
                sl = pl.ds(h * 16, 16)
                ov[e, sl] = exv[h] * (ov[e, sl] + vr[e, sl])
            return 0
        lax.fori_loop(0, G2, ebody, 0)

    issue_in(0, 0)
    issue_in(1, 1)

    def pair(g2, _):
        g = g2 * 2
        for b in (0, 1):
            gb = g + b
            wait_in(gb, b)
            compute(b)
            for cp in scatter_copies(b):
                cp.start(add=True)
            for cp in scatter_copies(b):
                cp.wait()
            @pl.when(gb < NG2 - 2)
            def _():
                issue_in(gb + 2, b)
        return 0
    lax.fori_loop(0, NG2 // 2, pair, 0)
    plsc.subcore_barrier()

    @pl.when(s < 10)
    def _():
        pltpu.sync_copy(agg_sh.at[pl.ds(s * 1000, 1000), :],
                        pagg_hbm.at[c, pl.ds(s * 1000, 1000), :])
        pltpu.sync_copy(ssum_sh.at[pl.ds(s * 1000, 1000), :],
                        pssum_hbm.at[c, pl.ds(s * 1000, 1000), :])


def _run_sc2(vt, ev0, ex, ei):
    mesh = plsc.VectorSubcoreMesh(core_axis_name="c", subcore_axis_name="s")
    kern = functools.partial(
        pl.kernel,
        mesh=mesh,
        out_type=[
            jax.ShapeDtypeStruct((2, N, 16), jnp.float32),
            jax.ShapeDtypeStruct((2, N, ND), jnp.float32),
        ],
        scratch_types=[
            pltpu.VMEM((EPW,), jnp.int32),
            pltpu.VMEM((G2,), jnp.int32),
            pltpu.VMEM((G2,), jnp.int32),
            pltpu.VMEM((16, G2), jnp.float32),
            pltpu.VMEM((16, G2), jnp.float32),
            pltpu.VMEM((G2, ND), jnp.float32),
            pltpu.VMEM((G2, ND), jnp.float32),
            pltpu.VMEM((G2, ND), jnp.float32),
            pltpu.VMEM((G2, ND), jnp.float32),
            pltpu.VMEM((G2, 16), jnp.float32),
            pltpu.VMEM((G2, 16), jnp.float32),
            pltpu.VMEM((25, 128), jnp.float32),
            pltpu.VMEM((200, 16), jnp.float32),
            pltpu.VMEM_SHARED((N, ND), jnp.float32),
            pltpu.VMEM_SHARED((N, 16), jnp.float32),
            pltpu.SemaphoreType.DMA,
            pltpu.SemaphoreType.DMA,
            pltpu.SemaphoreType.DMA,
            pltpu.SemaphoreType.DMA,
        ],
        compiler_params=_SC_PARAMS,
    )(_sc2_body)
    return kern(vt, ev0, ex, ei)


def _combine_body(pagg_ref, ps_ref, emat_ref, wo_ref, wob_ref, out_ref):
    agg = pagg_ref[0] + pagg_ref[1]
    ssum = (ps_ref[0] + ps_ref[1])[:, 0:H]
    recip = 1.0 / (ssum + 1e-16)
    scl = jnp.dot(recip, emat_ref[...], preferred_element_type=jnp.float32)
    dn = (((1,), (1,)), ((), ()))
    out_ref[...] = lax.dot_general(
        agg * scl, wo_ref[...], dn, preferred_element_type=jnp.float32
    ) + wob_ref[...]


def _combine(pagg, pssum, emat, wo, wob):
    return pl.pallas_call(
        _combine_body,
        grid=(N // NB,),
        in_specs=[
            pl.BlockSpec((2, NB, ND), lambda i: (0, i, 0)),
            pl.BlockSpec((2, NB, 16), lambda i: (0, i, 0)),
            pl.BlockSpec((H, ND), lambda i: (0, 0)),
            pl.BlockSpec((ND, ND), lambda i: (0, 0)),
            pl.BlockSpec((1, ND), lambda i: (0, 0)),
        ],
        out_specs=pl.BlockSpec((NB, ND), lambda i: (i, 0)),
        out_shape=jax.ShapeDtypeStruct((N, ND), jnp.float32),
    )(pagg, pssum, emat, wo, wob)


def _blockdiag(w):
    m = jnp.zeros((ND, ND), jnp.float32)
    for h in range(H):
        m = m.at[h * 16:(h + 1) * 16, h * 16:(h + 1) * 16].set(
            w[h * 16:(h + 1) * 16, :])
    return m


def kernel(node_tensors, edge_tensors, edge_index, Wnq_w, Wnq_b, Wnk_w,
           Wnk_b, Wnv_w, Wnv_b, Weq_w, Weq_b, Wek_w, Wek_b, Wev_w, Wev_b,
           Wo_w, Wo_b):
    bq = (Wnq_b + Weq_b).reshape(1, ND)
    bk = (Wnk_b + Wek_b).reshape(1, ND)
    bv = (Wnv_b + Wev_b).reshape(1, ND)
    bkd = _blockdiag(Wek_w)
    bqd = _blockdiag(Weq_w)
    ones8 = jnp.zeros((ND, H), jnp.float32)
    for h in range(H):
        ones8 = ones8.at[h * 16:(h + 1) * 16, h].set(1.0)
    emat = ones8.T

    eye16 = jnp.eye(ED, dtype=jnp.float32)
    a, b, vt = _make_tables(node_tensors, Wnq_w, Wnk_w, Wnv_w, bkd, bqd,
                            bq, bk, bv)
    t1t, et, ev0 = _make_t1_ev0(edge_tensors, Weq_w, Wek_w, Wev_w, ones8,
                                eye16)
    ex = _run_sc1(a, b, et, t1t, edge_index)
    pssum, pagg = _run_sc2(vt, ev0, ex, edge_index)
    return _combine(pagg, pssum, emat, Wo_w, Wo_b.reshape(1, ND))


# speedup vs baseline: 1.3161x; 1.3161x over previous
import functools
import math

import jax
import jax.numpy as jnp
from jax import lax
from jax.experimental import pallas as pl
from jax.experimental.pallas import tpu as pltpu
from jax.experimental.pallas import tpu_sc as plsc

N = 10000
E = 320000
ND = 128
ED = 16
H = 8
D = 16
SCALE = 1.0 / math.sqrt(D)

NB = 1000
EB = 2560
G1 = 80
NG1 = 125
G2 = 40
NG2 = 250
NW = 32
EPW = E // NW

_SC_PARAMS = pltpu.CompilerParams(
    needs_layout_passes=False, use_tc_tiling_on_sc=False)


def _tables_body(node_ref, wnq_ref, wnk_ref, wnv_ref, bkd_ref, bqd_ref,
                 bq_ref, bk_ref, bv_ref, a_ref, b_ref, v_ref):
    x = node_ref[...]
    dn = (((1,), (1,)), ((), ()))
    at = lax.dot_general(x, wnq_ref[...], dn,
                         preferred_element_type=jnp.float32) + bq_ref[...]
    bt = lax.dot_general(x, wnk_ref[...], dn,
                         preferred_element_type=jnp.float32) + bk_ref[...]
    vt = lax.dot_general(x, wnv_ref[...], dn,
                         preferred_element_type=jnp.float32) + bv_ref[...]
    ap = jnp.dot(at, bkd_ref[...], preferred_element_type=jnp.float32)
    bp = jnp.dot(bt, bqd_ref[...], preferred_element_type=jnp.float32)
    a_ref[:, :ND] = at
    a_ref[:, ND:] = ap
    b_ref[:, :ND] = bt
    b_ref[:, ND:] = bp
    v_ref[...] = vt


def _make_tables(node, wnq, wnk, wnv, bkd, bqd, bq, bk, bv):
    full = lambda shape: pl.BlockSpec(shape, lambda i: (0,) * len(shape))
    return pl.pallas_call(
        _tables_body,
        grid=(N // NB,),
        in_specs=[
            pl.BlockSpec((NB, ND), lambda i: (i, 0)),
            full((ND, ND)), full((ND, ND)), full((ND, ND)),
            full((ND, ND)), full((ND, ND)),
            full((1, ND)), full((1, ND)), full((1, ND)),
        ],
        out_specs=[
            pl.BlockSpec((NB, 2 * ND), lambda i: (i, 0)),
            pl.BlockSpec((NB, 2 * ND), lambda i: (i, 0)),
            pl.BlockSpec((NB, ND), lambda i: (i, 0)),
        ],
        out_shape=[
            jax.ShapeDtypeStruct((N, 2 * ND), jnp.float32),
            jax.ShapeDtypeStruct((N, 2 * ND), jnp.float32),
            jax.ShapeDtypeStruct((N, ND), jnp.float32),
        ],
    )(node, wnq, wnk, wnv, bkd, bqd, bq, bk, bv)


def _t1_body(edge_ref, weq_ref, wek_ref, wev_ref, ones_ref, eye_ref,
             t1t_ref, et_ref, ev0_ref):
    x = edge_ref[...]
    dn = (((1,), (1,)), ((), ()))
    eq = lax.dot_general(x, weq_ref[...], dn,
                         preferred_element_type=jnp.float32)
    ek = lax.dot_general(x, wek_ref[...], dn,
                         preferred_element_type=jnp.float32)
    t1t_ref[...] = lax.dot_general(ones_ref[...], eq * ek,
                                   (((0,), (1,)), ((), ())),
                                   preferred_element_type=jnp.float32)
    et_ref[...] = lax.dot_general(eye_ref[...], x,
                                  (((1,), (1,)), ((), ())),
                                  preferred_element_type=jnp.float32)
    ev0_ref[...] = lax.dot_general(x, wev_ref[...], dn,
                                   preferred_element_type=jnp.float32)


def _make_t1_ev0(edge, weq, wek, wev, ones8, eye16):
    return pl.pallas_call(
        _t1_body,
        grid=(E // EB,),
        in_specs=[
            pl.BlockSpec((EB, ED), lambda i: (i, 0)),
            pl.BlockSpec((ND, ED), lambda i: (0, 0)),
            pl.BlockSpec((ND, ED), lambda i: (0, 0)),
            pl.BlockSpec((ND, ED), lambda i: (0, 0)),
            pl.BlockSpec((ND, H), lambda i: (0, 0)),
            pl.BlockSpec((ED, ED), lambda i: (0, 0)),
        ],
        out_specs=[
            pl.BlockSpec((H, EB), lambda i: (0, i)),
            pl.BlockSpec((ED, EB), lambda i: (0, i)),
            pl.BlockSpec((EB, ND), lambda i: (i, 0)),
        ],
        out_shape=[
            jax.ShapeDtypeStruct((H, E), jnp.float32),
            jax.ShapeDtypeStruct((ED, E), jnp.float32),
            jax.ShapeDtypeStruct((E, ND), jnp.float32),
        ],
    )(edge, weq, wek, wev, ones8, eye16)


def _sc1_body(a_hbm, b_hbm, et_hbm, t1t_hbm, ei_hbm,
              ex_hbm,
              src_all, dst_all,
              a_rows0, a_rows1, b_rows0, b_rows1, edge_v0, edge_v1,
              t1_v0, t1_v1, ex_v0, ex_v1,
              sem_i0, sem_i1, sem_o0, sem_o1):
    c = lax.axis_index("c")
    s = lax.axis_index("s")
    base = (c * 16 + s) * EPW
    iot = jnp.arange(16, dtype=jnp.int32)
    bufs = ((a_rows0, b_rows0, edge_v0, t1_v0, ex_v0, sem_i0, sem_o0),
            (a_rows1, b_rows1, edge_v1, t1_v1, ex_v1, sem_i1, sem_o1))

    pltpu.sync_copy(ei_hbm.at[0, pl.ds(base, EPW)], src_all)
    pltpu.sync_copy(ei_hbm.at[1, pl.ds(base, EPW)], dst_all)

    def in_copies(g, b):
        ar, br, ev, tv, _, si, _ = bufs[b]
        off = base + g * G1
        return (
            pltpu.make_async_copy(et_hbm.at[:, pl.ds(off, G1)], ev, si),
            pltpu.make_async_copy(t1t_hbm.at[:, pl.ds(off, G1)], tv, si),
            pltpu.make_async_copy(a_hbm.at[src_all.at[pl.ds(g * G1, G1)]],
                                  ar, si),
            pltpu.make_async_copy(b_hbm.at[dst_all.at[pl.ds(g * G1, G1)]],
                                  br, si),
        )

    def issue_in(g, b):
        for cp in in_copies(g, b):
            cp.start()

    def wait_in(g, b):
        for cp in in_copies(g, b):
            cp.wait()

    def out_copy(g, b):
        xv, so = bufs[b][4], bufs[b][6]
        off = base + g * G1
        return pltpu.make_async_copy(xv, ex_hbm.at[pl.ds(off, G1), :], so)

    def compute(b):
        ar, br, ev, tv, xv = bufs[b][:5]

        def one_edge(e):
            esp = jnp.full((16,), e, jnp.int32)
            edg = plsc.load_gather(ev, [iot, esp])
            sums = []
            for h in range(H):
                qh = ar[e, pl.ds(h * 16, 16)]
                kh = br[e, pl.ds(h * 16, 16)]
                ph = (ar[e, pl.ds(ND + h * 16, 16)]
                      + br[e, pl.ds(ND + h * 16, 16)])
                sums.append(jnp.sum(qh * kh + ph * edg))
            acc = jnp.zeros((16,), jnp.float32)
            for h in range(H):
                acc = acc + jnp.where(iot == h, sums[h], jnp.float32(0.0))
            t1g = plsc.load_gather(tv, [iot & 7, esp])
            xv[e, :] = jnp.exp((acc + t1g) * SCALE)

        def ebody(p, _):
            one_edge(p * 2)
            one_edge(p * 2 + 1)
            return 0
        lax.fori_loop(0, G1 // 2, ebody, 0)

    issue_in(0, 0)
    issue_in(1, 1)

    def pair(g2, _):
        g = g2 * 2
        for b in (0, 1):
            gb = g + b
            wait_in(gb, b)
            @pl.when(gb >= 2)
            def _():
                out_copy(gb - 2, b).wait()
            compute(b)
            out_copy(gb, b).start()
            @pl.when(gb < NG1 - 2)
            def _():
                issue_in(gb + 2, b)
        return 0
    lax.fori_loop(0, (NG1 - 1) // 2, pair, 0)

    gb = NG1 - 1
    wait_in(gb, 0)
    out_copy(gb - 2, 0).wait()
    compute(0)
    out_copy(gb, 0).start()
    out_copy(gb - 1, 1).wait()
    out_copy(gb, 0).wait()


def _run_sc1(a, b, et, t1t, ei):
    mesh = plsc.VectorSubcoreMesh(core_axis_name="c", subcore_axis_name="s")
    kern = functools.partial(
        pl.kernel,
        mesh=mesh,
        out_type=jax.ShapeDtypeStruct((E, 16), jnp.float32),
        scratch_types=[
            pltpu.VMEM((EPW,), jnp.int32),
            pltpu.VMEM((EPW,), jnp.int32),
            pltpu.VMEM((G1, 2 * ND), jnp.float32),
            pltpu.VMEM((G1, 2 * ND), jnp.float32),
            pltpu.VMEM((G1, 2 * ND), jnp.float32),
            pltpu.VMEM((G1, 2 * ND), jnp.float32),
            pltpu.VMEM((ED, G1), jnp.float32),
            pltpu.VMEM((ED, G1), jnp.float32),
            pltpu.VMEM((H, G1), jnp.float32),
            pltpu.VMEM((H, G1), jnp.float32),
            pltpu.VMEM((G1, 16), jnp.float32),
            pltpu.VMEM((G1, 16), jnp.float32),
            pltpu.SemaphoreType.DMA,
            pltpu.SemaphoreType.DMA,
            pltpu.SemaphoreType.DMA,
            pltpu.SemaphoreType.DMA,
        ],
        compiler_params=_SC_PARAMS,
    )(_sc1_body)
    return kern(a, b, et, t1t, ei)


def _sc2_body(vt_hbm, ev0_hbm, ex_hbm, ei_hbm,
              pssum_hbm, pagg_hbm,
              dst_all,
              src_v0, src_v1, ex_v0, ex_v1, ev0_v0, ev0_v1,
              vt_rows0, vt_rows1, ss_v0, ss_v1, zb128, zb16,
              agg_sh, ssum_sh,
              sem_i0, sem_i1, sem_s0, sem_s1):
    c = lax.axis_index("c")
    s = lax.axis_index("s")
    base = (c * 16 + s) * EPW
    iot = jnp.arange(16, dtype=jnp.int32)
    bufs = ((src_v0, ex_v0, ev0_v0, vt_rows0, ss_v0, sem_i0, sem_s0),
            (src_v1, ex_v1, ev0_v1, vt_rows1, ss_v1, sem_i1, sem_s1))

    pltpu.sync_copy(ei_hbm.at[1, pl.ds(base, EPW)], dst_all)

    zero16 = jnp.zeros((16,), jnp.float32)
    def zr(i, _):
        for j in range(8):
            zb128[i, pl.ds(j * 16, 16)] = zero16
        zb16[i, :] = zero16
        return 0
    lax.fori_loop(0, 25, zr, 0)
    def zr2(i, _):
        zb16[i, :] = zero16
        return 0
    lax.fori_loop(25, 200, zr2, 0)
    @pl.when(s < 10)
    def _():
        for k in range(40):
            pltpu.sync_copy(zb128,
                            agg_sh.at[pl.ds(s * 1000 + k * 25, 25), :])
        for k in range(5):
            pltpu.sync_copy(zb16,
                            ssum_sh.at[pl.ds(s * 1000 + k * 200, 200), :])
    plsc.subcore_barrier()

    def in_copies(g, b):
        sv, xv, ov, vr, _, si, _ = bufs[b]
        off = base + g * G2
        return (
            pltpu.make_async_copy(ei_hbm.at[0, pl.ds(off, G2)], sv, si),
            pltpu.make_async_copy(ex_hbm.at[pl.ds(off, G2), :], xv, si),
            pltpu.make_async_copy(ev0_hbm.at[pl.ds(off, G2), :], ov, si),
            pltpu.make_async_copy(vt_hbm.at[dst_all.at[pl.ds(g * G2, G2)]],
                                  vr, si),
        )

    def issue_in(g, b):
        for cp in in_copies(g, b):
            cp.start()

    def wait_in(g, b):
        for cp in in_copies(g, b):
            cp.wait()

    def scatter_copies(b):
        sv, xv, ov = bufs[b][:3]
        ss = bufs[b][6]
        return (
            pltpu.make_async_copy(ov, agg_sh.at[sv], ss),
            pltpu.make_async_copy(xv, ssum_sh.at[sv], ss),
        )

    def compute(b):
        _, xv, ov, vr = bufs[b][:4]

        def ebody(e, _):
            exv = xv[e, :]
            for h in range(H):
                sl = pl.ds(h * 16, 16)
                ov[e, sl] = exv[h] * (ov[e, sl] + vr[e, sl])
            return 0
        lax.fori_loop(0, G2, ebody, 0)

    issue_in(0, 0)
    issue_in(1, 1)

    def pair(g2, _):
        g = g2 * 2
        for b in (0, 1):
            gb = g + b
            wait_in(gb, b)
            compute(b)
            for cp in scatter_copies(b):
                cp.start(add=True)
            for cp in scatter_copies(b):
                cp.wait()
            @pl.when(gb < NG2 - 2)
            def _():
                issue_in(gb + 2, b)
        return 0
    lax.fori_loop(0, NG2 // 2, pair, 0)
    plsc.subcore_barrier()

    @pl.when(s < 10)
    def _():
        pltpu.sync_copy(agg_sh.at[pl.ds(s * 1000, 1000), :],
                        pagg_hbm.at[c, pl.ds(s * 1000, 1000), :])
        pltpu.sync_copy(ssum_sh.at[pl.ds(s * 1000, 1000), :],
                        pssum_hbm.at[c, pl.ds(s * 1000, 1000), :])


def _run_sc2(vt, ev0, ex, ei):
    mesh = plsc.VectorSubcoreMesh(core_axis_name="c", subcore_axis_name="s")
    kern = functools.partial(
        pl.kernel,
        mesh=mesh,
        out_type=[
            jax.ShapeDtypeStruct((2, N, 16), jnp.float32),
            jax.ShapeDtypeStruct((2, N, ND), jnp.float32),
        ],
        scratch_types=[
            pltpu.VMEM((EPW,), jnp.int32),
            pltpu.VMEM((G2,), jnp.int32),
            pltpu.VMEM((G2,), jnp.int32),
            pltpu.VMEM((G2, 16), jnp.float32),
            pltpu.VMEM((G2, 16), jnp.float32),
            pltpu.VMEM((G2, ND), jnp.float32),
            pltpu.VMEM((G2, ND), jnp.float32),
            pltpu.VMEM((G2, ND), jnp.float32),
            pltpu.VMEM((G2, ND), jnp.float32),
            pltpu.VMEM((G2, 16), jnp.float32),
            pltpu.VMEM((G2, 16), jnp.float32),
            pltpu.VMEM((25, 128), jnp.float32),
            pltpu.VMEM((200, 16), jnp.float32),
            pltpu.VMEM_SHARED((N, ND), jnp.float32),
            pltpu.VMEM_SHARED((N, 16), jnp.float32),
            pltpu.SemaphoreType.DMA,
            pltpu.SemaphoreType.DMA,
            pltpu.SemaphoreType.DMA,
            pltpu.SemaphoreType.DMA,
        ],
        compiler_params=_SC_PARAMS,
    )(_sc2_body)
    return kern(vt, ev0, ex, ei)


def _combine_body(pagg_ref, ps_ref, emat_ref, wo_ref, wob_ref, out_ref):
    agg = pagg_ref[0] + pagg_ref[1]
    ssum = (ps_ref[0] + ps_ref[1])[:, 0:H]
    recip = 1.0 / (ssum + 1e-16)
    scl = jnp.dot(recip, emat_ref[...], preferred_element_type=jnp.float32)
    dn = (((1,), (1,)), ((), ()))
    out_ref[...] = lax.dot_general(
        agg * scl, wo_ref[...], dn, preferred_element_type=jnp.float32
    ) + wob_ref[...]


def _combine(pagg, pssum, emat, wo, wob):
    return pl.pallas_call(
        _combine_body,
        grid=(N // NB,),
        in_specs=[
            pl.BlockSpec((2, NB, ND), lambda i: (0, i, 0)),
            pl.BlockSpec((2, NB, 16), lambda i: (0, i, 0)),
            pl.BlockSpec((H, ND), lambda i: (0, 0)),
            pl.BlockSpec((ND, ND), lambda i: (0, 0)),
            pl.BlockSpec((1, ND), lambda i: (0, 0)),
        ],
        out_specs=pl.BlockSpec((NB, ND), lambda i: (i, 0)),
        out_shape=jax.ShapeDtypeStruct((N, ND), jnp.float32),
    )(pagg, pssum, emat, wo, wob)


def _blockdiag(w):
    m = jnp.zeros((ND, ND), jnp.float32)
    for h in range(H):
        m = m.at[h * 16:(h + 1) * 16, h * 16:(h + 1) * 16].set(
            w[h * 16:(h + 1) * 16, :])
    return m


def kernel(node_tensors, edge_tensors, edge_index, Wnq_w, Wnq_b, Wnk_w,
           Wnk_b, Wnv_w, Wnv_b, Weq_w, Weq_b, Wek_w, Wek_b, Wev_w, Wev_b,
           Wo_w, Wo_b):
    bq = (Wnq_b + Weq_b).reshape(1, ND)
    bk = (Wnk_b + Wek_b).reshape(1, ND)
    bv = (Wnv_b + Wev_b).reshape(1, ND)
    bkd = _blockdiag(Wek_w)
    bqd = _blockdiag(Weq_w)
    ones8 = jnp.zeros((ND, H), jnp.float32)
    for h in range(H):
        ones8 = ones8.at[h * 16:(h + 1) * 16, h].set(1.0)
    emat = ones8.T

    eye16 = jnp.eye(ED, dtype=jnp.float32)
    a, b, vt = _make_tables(node_tensors, Wnq_w, Wnk_w, Wnv_w, bkd, bqd,
                            bq, bk, bv)
    t1t, et, ev0 = _make_t1_ev0(edge_tensors, Weq_w, Wek_w, Wev_w, ones8,
                                eye16)
    ex = _run_sc1(a, b, et, t1t, edge_index)
    pssum, pagg = _run_sc2(vt, ev0, ex, edge_index)
    return _combine(pagg, pssum, emat, Wo_w, Wo_b.reshape(1, ND))
